# Initial kernel scaffold; baseline (speedup 1.0000x reference)
import jax
import jax.numpy as jnp
from jax.experimental import pallas as pl


def kernel(inputs):
    raise NotImplementedError("write your pallas kernel here")


# speedup vs baseline: 4.9664x; 4.9664x over previous
import functools

import numpy as np

import jax
import jax.numpy as jnp
from jax import lax
from jax.experimental import pallas as pl
from jax.experimental.pallas import tpu as pltpu
from jax.experimental.pallas import tpu_sc as plsc

K_TOP = 64
NC, NS, L = 2, 16, 16
NW = NC * NS
B, N, F = 4, 8192, 2048
FPW = F // NW
NG = FPW // L
NTASK = B * NG
CHUNK = 2048
NCHUNK = N // CHUNK
UNROLL = 4
CAP = 2048
NBKT = 256

_MASK7F = np.int32(0x7FFFFFFF)


def _flip(xi):
    return lax.bitwise_xor(xi, lax.bitwise_and(lax.shift_right_arithmetic(xi, 31), _MASK7F))


def _bcast(x, dtype=jnp.int32):
    return lax.broadcast(lax.convert_element_type(x, dtype), (L,))


def _ivec(v):
    return _bcast(np.int32(v))


def _scan_desc(hist, target):
    def body(i, carry):
        run, p, ca = carry
        bkt = NBKT - 1 - i
        h = hist[bkt]
        run2 = run + h
        newf = jnp.logical_and(run2 >= target, run < target)
        p = jnp.where(newf, _bcast(bkt), p)
        ca = jnp.where(newf, run, ca)
        return run2, p, ca
    z = _ivec(0)
    _, p, ca = lax.fori_loop(0, NBKT, body, (z, z, z))
    return p, ca


def _clear_hist(hist):
    z = _ivec(0)
    def body(i, _):
        hist[i] = z
        return 0
    lax.fori_loop(0, NBKT, body, 0)


def _kernel_body(in_hbm, out_hbm, buf, cand, hist, outv):
    cid = lax.axis_index("c")
    sid = lax.axis_index("s")
    wid = sid * NC + cid
    lane = lax.iota(jnp.int32, L)
    ones = _ivec(1)
    zero = _ivec(0)

    def task(t, _):
        g = lax.rem(t, NG)
        b = lax.div(t, NG)
        f0 = wid * FPW + g * L
        row0 = b * N

        _clear_hist(hist)

        def chunk1(c, _):
            pltpu.sync_copy(in_hbm.at[pl.ds(row0 + c * CHUNK, CHUNK), pl.ds(f0, L)], buf)

            def step(s, _):
                for u in range(UNROLL):
                    v = buf[s * UNROLL + u]
                    ks = _flip(plsc.bitcast(v, jnp.int32))
                    d0 = lax.bitwise_xor(lax.shift_right_logical(ks, 24), np.int32(128))
                    plsc.addupdate_scatter(hist, [d0, lane], ones)
                return 0
            lax.fori_loop(0, CHUNK // UNROLL, step, 0)
            return 0
        lax.fori_loop(0, NCHUNK, chunk1, 0)

        p1, ca0 = _scan_desc(hist, _bcast(K_TOP))

        def chunk2(c, ptr):
            pltpu.sync_copy(in_hbm.at[pl.ds(row0 + c * CHUNK, CHUNK), pl.ds(f0, L)], buf)

            def step(s, ptr):
                for u in range(UNROLL):
                    v = buf[s * UNROLL + u]
                    ks = _flip(plsc.bitcast(v, jnp.int32))
                    d0 = lax.bitwise_xor(lax.shift_right_logical(ks, 24), np.int32(128))
                    m = jnp.logical_and(d0 >= p1, ptr < CAP)
                    plsc.store_scatter(cand, [ptr, lane], ks, mask=m)
                    ptr = ptr + jnp.where(m, ones, zero)
                return ptr
            return lax.fori_loop(0, CHUNK // UNROLL, step, ptr)
        ncand = lax.fori_loop(0, NCHUNK, chunk2, zero)
        nmax = jnp.max(ncand)

        r = _bcast(K_TOP) - ca0
        pref = lax.bitwise_xor(p1, _bcast(128))
        for sh in (16, 8, 0):
            _clear_hist(hist)

            def rhist(i, _, sh=sh, pref=pref):
                ks = cand[i]
                valid = _bcast(i) < ncand
                match = jnp.logical_and(lax.shift_right_logical(ks, sh + 8) == pref, valid)
                d = lax.bitwise_and(lax.shift_right_logical(ks, sh), np.int32(0xFF))
                plsc.addupdate_scatter(hist, [d, lane], ones, mask=match)
                return 0
            lax.fori_loop(0, nmax, rhist, 0)
            p, ca = _scan_desc(hist, r)
            pref = lax.bitwise_or(lax.shift_left(pref, 8), p)
            r = r - ca

        ks_t = pref
        t_f = plsc.bitcast(_flip(ks_t), jnp.float32)

        def fill(i, _):
            outv[i] = t_f
            return 0
        lax.fori_loop(0, K_TOP, fill, 0)

        def coll(i, optr):
            ks = cand[i]
            valid = _bcast(i) < ncand
            m = jnp.logical_and(jnp.logical_and(ks > ks_t, valid), optr < K_TOP)
            v = plsc.bitcast(_flip(ks), jnp.float32)
            plsc.store_scatter(outv, [optr, lane], v, mask=m)
            return optr + jnp.where(m, ones, zero)
        lax.fori_loop(0, nmax, coll, zero)

        kk = 2
        while kk <= K_TOP:
            j = kk // 2
            while j >= 1:
                lg = j.bit_length() - 1

                def ce(q, _, j=j, lg=lg, kk=kk):
                    low = lax.bitwise_and(q, j - 1)
                    i = lax.bitwise_or(lax.shift_left(lax.shift_right_logical(q, lg), lg + 1), low)
                    l2 = lax.bitwise_or(i, j)
                    a = outv[i]
                    bb = outv[l2]
                    mx = jnp.maximum(a, bb)
                    mn = jnp.minimum(a, bb)
                    up = _bcast(lax.bitwise_and(i, kk)) == 0
                    outv[i] = jnp.where(up, mx, mn)
                    outv[l2] = jnp.where(up, mn, mx)
                    return 0
                lax.fori_loop(0, K_TOP // 2, ce, 0)
                j //= 2
            kk *= 2

        pltpu.sync_copy(outv, out_hbm.at[pl.ds(b * K_TOP, K_TOP), pl.ds(f0, L)])
        return 0

    lax.fori_loop(0, NTASK, task, 0)


@jax.jit
def _run(inputs2d):
    mesh = plsc.VectorSubcoreMesh(
        core_axis_name="c", subcore_axis_name="s", num_cores=NC, num_subcores=NS)
    f = pl.kernel(
        _kernel_body,
        out_type=jax.ShapeDtypeStruct((B * K_TOP, F), jnp.float32),
        mesh=mesh,
        compiler_params=pltpu.CompilerParams(use_tc_tiling_on_sc=False, needs_layout_passes=False),
        scratch_types=[
            pltpu.VMEM((CHUNK, L), jnp.float32),
            pltpu.VMEM((CAP, L), jnp.int32),
            pltpu.VMEM((NBKT, L), jnp.int32),
            pltpu.VMEM((K_TOP, L), jnp.float32),
        ],
    )
    return f(inputs2d)


def kernel(inputs):
    out2d = _run(inputs.reshape(B * N, F))
    return out2d.reshape(B, K_TOP, F)
